# baseline (speedup 1.0000x reference)
import functools

import jax
import jax.numpy as jnp
from jax import lax
from jax.experimental import pallas as pl
from jax.experimental.pallas import tpu as pltpu
from jax.experimental.pallas import tpu_sc as plsc

EDGE_DIM = 16
NUM_EDGE_TYPES = 4
MAX_POS = 32
LUT_ROWS = NUM_EDGE_TYPES * MAX_POS * 3
LUT_FLAT = LUT_ROWS * EDGE_DIM

B = 2560
G = B // 16
IN_BYTES = B * 4


@functools.lru_cache(maxsize=None)
def _make_sc_kernel(E):
    info = plsc.get_sparse_core_info()
    NC, NS = info.num_cores, info.num_subcores
    NW = NC * NS
    assert E % B == 0
    n_chunks = E // B
    n_iter = (n_chunks + NW - 1) // NW
    n_pairs = (n_iter + 1) // 2
    mesh = plsc.VectorSubcoreMesh(core_axis_name="c", subcore_axis_name="s")

    @functools.partial(
        pl.kernel,
        mesh=mesh,
        compiler_params=pltpu.CompilerParams(needs_layout_passes=False),
        out_type=jax.ShapeDtypeStruct((EDGE_DIM, E), jnp.float32),
        scratch_types=[
            pltpu.VMEM((NUM_EDGE_TYPES * EDGE_DIM,), jnp.float32),
            pltpu.VMEM((MAX_POS * EDGE_DIM,), jnp.float32),
            pltpu.VMEM((3 * EDGE_DIM,), jnp.float32),
            pltpu.VMEM((LUT_FLAT,), jnp.float32),
            pltpu.VMEM((2, B), jnp.int32),
            pltpu.VMEM((2, B), jnp.int32),
            pltpu.VMEM((2, B), jnp.int32),
            pltpu.VMEM((2, B), jnp.int32),
            pltpu.VMEM((2, EDGE_DIM, B), jnp.float32),
            pltpu.SemaphoreType.DMA,
            pltpu.SemaphoreType.DMA,
        ],
    )
    def sc_kernel(wt_hbm, wp_hbm, wb_hbm, t_hbm, p_hbm, blk_hbm,
                  out_hbm, wt_v, wp_v, wb_v, lut_v, t_v, p_v, b0_v, b1_v,
                  rows_v, in_sem, out_sem):
        wid = lax.axis_index("s") * NC + lax.axis_index("c")

        def chunk_off(i):
            return jnp.minimum(wid + i * NW, n_chunks - 1) * B

        def start_in(i, par):
            off = chunk_off(i)
            pltpu.async_copy(t_hbm.at[pl.ds(off, B)], t_v.at[par], in_sem)
            pltpu.async_copy(p_hbm.at[pl.ds(off, B)], p_v.at[par], in_sem)
            pltpu.async_copy(blk_hbm.at[0, pl.ds(off, B)], b0_v.at[par],
                             in_sem)
            pltpu.async_copy(blk_hbm.at[1, pl.ds(off, B)], b1_v.at[par],
                             in_sem)

        def wait_in(par):
            for buf in (t_v, p_v, b0_v, b1_v):
                pltpu.make_async_copy(t_hbm.at[pl.ds(0, B)], buf.at[par],
                                      in_sem).wait()

        def start_out(i, par):
            off = chunk_off(i)
            pltpu.async_copy(rows_v.at[par], out_hbm.at[:, pl.ds(off, B)],
                             out_sem)

        def wait_out(par):
            pltpu.make_async_copy(rows_v.at[par],
                                  out_hbm.at[:, pl.ds(0, B)], out_sem).wait()

        start_in(0, 0)
        start_in(1, 1)

        pltpu.sync_copy(wt_hbm, wt_v)
        pltpu.sync_copy(wp_hbm, wp_v)
        pltpu.sync_copy(wb_hbm, wb_v)
        wb_rows = [wb_v[pl.ds(s * EDGE_DIM, EDGE_DIM)] for s in range(3)]
        lane = lax.iota(jnp.int32, 16)
        for t in range(NUM_EDGE_TYPES):
            wt_row = wt_v[pl.ds(t * EDGE_DIM, EDGE_DIM)]
            for p in range(MAX_POS):
                tp = wt_row + wp_v[pl.ds(p * EDGE_DIM, EDGE_DIM)]
                base = (t * MAX_POS + p) * 3 * EDGE_DIM
                for s in range(3):
                    lut_v[pl.ds(base + s * EDGE_DIM, EDGE_DIM)] = (
                        tp + wb_rows[s])

        zero = jnp.zeros((16,), jnp.int32)
        rotc = [(lane + j) & (EDGE_DIM - 1) for j in range(EDGE_DIM)]

        def compute(par):
            tb, pb, b0b, b1b = (t_v, p_v, b0_v, b1_v)
            rb = rows_v.at[par]

            @plsc.parallel_loop(0, G, unroll=4)
            def _group(g):
                sl = pl.ds(g * 16, 16)
                t = tb[par, sl]
                p = jnp.minimum(jnp.maximum(pb[par, sl], zero),
                                zero + (MAX_POS - 1))
                d = b0b[par, sl] - b1b[par, sl]
                s = jnp.where(d > 0, zero + 2,
                              jnp.where(d < 0, zero, zero + 1))
                addr = (t * (MAX_POS * 3) + p * 3 + s) * EDGE_DIM
                ecol = lane + g * 16
                for j in range(EDGE_DIM):
                    vals = plsc.load_gather(lut_v, [addr + rotc[j]])
                    plsc.store_scatter(rb, [rotc[j], ecol], vals)

        def pair(i2, carry):
            for par in (0, 1):
                i = i2 * 2 + par

                @pl.when(wid + i * NW < n_chunks)
                def _step():
                    wait_in(par)

                    @pl.when(i2 > 0)
                    def _():
                        wait_out(par)

                    compute(par)
                    start_out(i, par)

                    @pl.when(wid + (i + 2) * NW < n_chunks)
                    def _():
                        start_in(i + 2, par)
            return carry

        lax.fori_loop(0, n_pairs, pair, 0)
        wait_out(0)
        wait_out(1)

    return sc_kernel


def kernel(edge_types, edge_pos, block_idx, W_type, W_pos, W_blk):
    E = edge_types.shape[0]
    sc = _make_sc_kernel(E)
    out_t = sc(W_type.reshape(-1), W_pos.reshape(-1), W_blk.reshape(-1),
               edge_types, edge_pos, block_idx)
    return out_t.T


# speedup vs baseline: 2.3019x; 2.3019x over previous
import functools

import jax
import jax.numpy as jnp
from jax import lax
from jax.experimental import pallas as pl
from jax.experimental.pallas import tpu as pltpu
from jax.experimental.pallas import tpu_sc as plsc

EDGE_DIM = 16
NUM_EDGE_TYPES = 4
MAX_POS = 32
LUT_ROWS = NUM_EDGE_TYPES * MAX_POS * 3
LUT_STRIDE = EDGE_DIM + 1
LUT_FLAT = LUT_ROWS * LUT_STRIDE

B = 2560
G = B // 16
IN_BYTES = B * 4


@functools.lru_cache(maxsize=None)
def _make_sc_kernel(E):
    info = plsc.get_sparse_core_info()
    NC, NS = info.num_cores, info.num_subcores
    NW = NC * NS
    assert E % B == 0
    n_chunks = E // B
    n_iter = (n_chunks + NW - 1) // NW
    n_pairs = (n_iter + 1) // 2
    mesh = plsc.VectorSubcoreMesh(core_axis_name="c", subcore_axis_name="s")

    @functools.partial(
        pl.kernel,
        mesh=mesh,
        compiler_params=pltpu.CompilerParams(needs_layout_passes=False),
        out_type=jax.ShapeDtypeStruct((EDGE_DIM, E), jnp.float32),
        scratch_types=[
            pltpu.VMEM((NUM_EDGE_TYPES * EDGE_DIM,), jnp.float32),
            pltpu.VMEM((MAX_POS * EDGE_DIM,), jnp.float32),
            pltpu.VMEM((3 * EDGE_DIM,), jnp.float32),
            pltpu.VMEM((LUT_FLAT,), jnp.float32),
            pltpu.VMEM((2, B), jnp.int32),
            pltpu.VMEM((2, B), jnp.int32),
            pltpu.VMEM((2, B), jnp.int32),
            pltpu.VMEM((2, B), jnp.int32),
            pltpu.VMEM((2, EDGE_DIM, B), jnp.float32),
            pltpu.SemaphoreType.DMA,
            pltpu.SemaphoreType.DMA,
        ],
    )
    def sc_kernel(wt_hbm, wp_hbm, wb_hbm, t_hbm, p_hbm, blk_hbm,
                  out_hbm, wt_v, wp_v, wb_v, lut_v, t_v, p_v, b0_v, b1_v,
                  rows_v, in_sem, out_sem):
        wid = lax.axis_index("s") * NC + lax.axis_index("c")

        def chunk_off(i):
            return jnp.minimum(wid + i * NW, n_chunks - 1) * B

        def start_in(i, par):
            off = chunk_off(i)
            pltpu.async_copy(t_hbm.at[pl.ds(off, B)], t_v.at[par], in_sem)
            pltpu.async_copy(p_hbm.at[pl.ds(off, B)], p_v.at[par], in_sem)
            pltpu.async_copy(blk_hbm.at[0, pl.ds(off, B)], b0_v.at[par],
                             in_sem)
            pltpu.async_copy(blk_hbm.at[1, pl.ds(off, B)], b1_v.at[par],
                             in_sem)

        def wait_in(par):
            for buf in (t_v, p_v, b0_v, b1_v):
                pltpu.make_async_copy(t_hbm.at[pl.ds(0, B)], buf.at[par],
                                      in_sem).wait()

        def start_out(i, par):
            off = chunk_off(i)
            pltpu.async_copy(rows_v.at[par], out_hbm.at[:, pl.ds(off, B)],
                             out_sem)

        def wait_out(par):
            pltpu.make_async_copy(rows_v.at[par],
                                  out_hbm.at[:, pl.ds(0, B)], out_sem).wait()

        start_in(0, 0)
        start_in(1, 1)

        pltpu.sync_copy(wt_hbm, wt_v)
        pltpu.sync_copy(wp_hbm, wp_v)
        pltpu.sync_copy(wb_hbm, wb_v)
        wb_rows = [wb_v[pl.ds(s * EDGE_DIM, EDGE_DIM)] for s in range(3)]
        lane = lax.iota(jnp.int32, 16)
        for t in range(NUM_EDGE_TYPES):
            wt_row = wt_v[pl.ds(t * EDGE_DIM, EDGE_DIM)]
            for p in range(MAX_POS):
                tp = wt_row + wp_v[pl.ds(p * EDGE_DIM, EDGE_DIM)]
                for s in range(3):
                    r = (t * MAX_POS + p) * 3 + s
                    plsc.store_scatter(lut_v, [lane + r * LUT_STRIDE],
                                       tp + wb_rows[s])

        zero = jnp.zeros((16,), jnp.int32)

        def compute(par):
            tb, pb, b0b, b1b, rb = (t_v, p_v, b0_v, b1_v, rows_v)

            @plsc.parallel_loop(0, G, unroll=2)
            def _group(g):
                sl = pl.ds(g * 16, 16)
                t = tb[par, sl]
                p = jnp.minimum(jnp.maximum(pb[par, sl], zero),
                                zero + (MAX_POS - 1))
                d = b0b[par, sl] - b1b[par, sl]
                s = jnp.where(d > 0, zero + 2,
                              jnp.where(d < 0, zero, zero + 1))
                addr = (t * (MAX_POS * 3) + p * 3 + s) * LUT_STRIDE
                for j in range(EDGE_DIM):
                    rb[par, j, sl] = plsc.load_gather(lut_v, [addr + j])

        def pair(i2, carry):
            for par in (0, 1):
                i = i2 * 2 + par

                @pl.when(wid + i * NW < n_chunks)
                def _step():
                    wait_in(par)

                    @pl.when(i2 > 0)
                    def _():
                        wait_out(par)

                    compute(par)
                    start_out(i, par)

                    @pl.when(wid + (i + 2) * NW < n_chunks)
                    def _():
                        start_in(i + 2, par)
            return carry

        lax.fori_loop(0, n_pairs, pair, 0)
        wait_out(0)
        wait_out(1)

    return sc_kernel


def kernel(edge_types, edge_pos, block_idx, W_type, W_pos, W_blk):
    E = edge_types.shape[0]
    sc = _make_sc_kernel(E)
    out_t = sc(W_type.reshape(-1), W_pos.reshape(-1), W_blk.reshape(-1),
               edge_types, edge_pos, block_idx)
    return out_t.T


# speedup vs baseline: 2.3755x; 1.0320x over previous
import functools

import jax
import jax.numpy as jnp
from jax import lax
from jax.experimental import pallas as pl
from jax.experimental.pallas import tpu as pltpu
from jax.experimental.pallas import tpu_sc as plsc

EDGE_DIM = 16
NUM_EDGE_TYPES = 4
MAX_POS = 32
LUT_ROWS = NUM_EDGE_TYPES * MAX_POS * 3
LUT_STRIDE = EDGE_DIM + 1
LUT_FLAT = LUT_ROWS * LUT_STRIDE

B = 2560
G = B // 16
IN_BYTES = B * 4


@functools.lru_cache(maxsize=None)
def _make_sc_kernel(E):
    info = plsc.get_sparse_core_info()
    NC, NS = info.num_cores, info.num_subcores
    NW = NC * NS
    assert E % B == 0
    n_chunks = E // B
    n_iter = (n_chunks + NW - 1) // NW
    n_pairs = (n_iter + 1) // 2
    mesh = plsc.VectorSubcoreMesh(core_axis_name="c", subcore_axis_name="s")

    @functools.partial(
        pl.kernel,
        mesh=mesh,
        compiler_params=pltpu.CompilerParams(needs_layout_passes=False),
        out_type=jax.ShapeDtypeStruct((EDGE_DIM, E), jnp.float32),
        scratch_types=[
            pltpu.VMEM((NUM_EDGE_TYPES * EDGE_DIM,), jnp.float32),
            pltpu.VMEM((MAX_POS * EDGE_DIM,), jnp.float32),
            pltpu.VMEM((3 * EDGE_DIM,), jnp.float32),
            pltpu.VMEM((LUT_FLAT,), jnp.float32),
            pltpu.VMEM((2, B), jnp.int32),
            pltpu.VMEM((2, B), jnp.int32),
            pltpu.VMEM((2, B), jnp.int32),
            pltpu.VMEM((2, B), jnp.int32),
            pltpu.VMEM((2, EDGE_DIM, B), jnp.float32),
            pltpu.SemaphoreType.DMA,
            pltpu.SemaphoreType.DMA,
        ],
    )
    def sc_kernel(wt_hbm, wp_hbm, wb_hbm, t_hbm, p_hbm, blk_hbm,
                  out_hbm, wt_v, wp_v, wb_v, lut_v, t_v, p_v, b0_v, b1_v,
                  rows_v, in_sem, out_sem):
        wid = lax.axis_index("s") * NC + lax.axis_index("c")

        def chunk_off(i):
            return jnp.minimum(wid + i * NW, n_chunks - 1) * B

        def start_in(i, par):
            off = chunk_off(i)
            pltpu.async_copy(t_hbm.at[pl.ds(off, B)], t_v.at[par], in_sem)
            pltpu.async_copy(p_hbm.at[pl.ds(off, B)], p_v.at[par], in_sem)
            pltpu.async_copy(blk_hbm.at[0, pl.ds(off, B)], b0_v.at[par],
                             in_sem)
            pltpu.async_copy(blk_hbm.at[1, pl.ds(off, B)], b1_v.at[par],
                             in_sem)

        def wait_in(par):
            for buf in (t_v, p_v, b0_v, b1_v):
                pltpu.make_async_copy(t_hbm.at[pl.ds(0, B)], buf.at[par],
                                      in_sem).wait()

        def start_out(i, par):
            off = chunk_off(i)
            pltpu.async_copy(rows_v.at[par], out_hbm.at[:, pl.ds(off, B)],
                             out_sem)

        def wait_out(par):
            pltpu.make_async_copy(rows_v.at[par],
                                  out_hbm.at[:, pl.ds(0, B)], out_sem).wait()

        start_in(0, 0)
        start_in(1, 1)

        pltpu.sync_copy(wt_hbm, wt_v)
        pltpu.sync_copy(wp_hbm, wp_v)
        pltpu.sync_copy(wb_hbm, wb_v)
        wb_rows = [wb_v[pl.ds(s * EDGE_DIM, EDGE_DIM)] for s in range(3)]
        lane = lax.iota(jnp.int32, 16)
        for t in range(NUM_EDGE_TYPES):
            wt_row = wt_v[pl.ds(t * EDGE_DIM, EDGE_DIM)]
            for p in range(MAX_POS):
                tp = wt_row + wp_v[pl.ds(p * EDGE_DIM, EDGE_DIM)]
                for s in range(3):
                    r = (t * MAX_POS + p) * 3 + s
                    plsc.store_scatter(lut_v, [lane + r * LUT_STRIDE],
                                       tp + wb_rows[s])

        zero = jnp.zeros((16,), jnp.int32)

        def compute(par):
            tb, pb, b0b, b1b, rb = (t_v, p_v, b0_v, b1_v, rows_v)

            @plsc.parallel_loop(0, G, unroll=1)
            def _group(g):
                sl = pl.ds(g * 16, 16)
                t = tb[par, sl]
                p = jnp.minimum(jnp.maximum(pb[par, sl], zero),
                                zero + (MAX_POS - 1))
                d = b0b[par, sl] - b1b[par, sl]
                s = jnp.where(d > 0, zero + 2,
                              jnp.where(d < 0, zero, zero + 1))
                addr = (t * (MAX_POS * 3) + p * 3 + s) * LUT_STRIDE
                for j in range(EDGE_DIM):
                    rb[par, j, sl] = plsc.load_gather(lut_v, [addr + j])

        def pair(i2, carry):
            for par in (0, 1):
                i = i2 * 2 + par

                @pl.when(wid + i * NW < n_chunks)
                def _step():
                    wait_in(par)

                    @pl.when(i2 > 0)
                    def _():
                        wait_out(par)

                    compute(par)
                    start_out(i, par)

                    @pl.when(wid + (i + 2) * NW < n_chunks)
                    def _():
                        start_in(i + 2, par)
            return carry

        lax.fori_loop(0, n_pairs, pair, 0)
        wait_out(0)
        wait_out(1)

    return sc_kernel


def kernel(edge_types, edge_pos, block_idx, W_type, W_pos, W_blk):
    E = edge_types.shape[0]
    sc = _make_sc_kernel(E)
    out_t = sc(W_type.reshape(-1), W_pos.reshape(-1), W_blk.reshape(-1),
               edge_types, edge_pos, block_idx)
    return out_t.T
